# baseline (speedup 1.0000x reference)
import jax
import jax.numpy as jnp
from jax import lax
from jax.experimental import pallas as pl
from jax.experimental.pallas import tpu as pltpu
from jax.experimental.pallas import tpu_sc as plsc

_D = 64
_B = 4096
_L = 200
_NC = 2
_NS = 16
_NW = _NC * _NS
_PER_W = _B // _NW
_IDX_MINOR = 100
_CHUNKS = _L // _IDX_MINOR


def _tec_body(idx_hbm, table_hbm, out_hbm, idx_v, rows, tbuf, gsem):
    wid = lax.axis_index("s") * _NC + lax.axis_index("c")
    b0 = wid * _PER_W

    pltpu.sync_copy(idx_hbm.at[pl.ds(b0 * _CHUNKS, _PER_W * _CHUNKS)], idx_v)

    iota = lax.iota(jnp.int32, 16)
    colbase = [(jnp.int32(g * 16) + iota) * jnp.int32(_L) for g in range(_D // 16)]

    @pl.loop(0, _PER_W)
    def _iter(i):
        c0 = pltpu.async_copy(
            table_hbm.at[idx_v.at[_CHUNKS * i]],
            rows.at[pl.ds(0, _IDX_MINOR)], gsem)
        c1 = pltpu.async_copy(
            table_hbm.at[idx_v.at[_CHUNKS * i + 1]],
            rows.at[pl.ds(_IDX_MINOR, _IDX_MINOR)], gsem)
        c0.wait()
        c1.wait()

        @pl.loop(0, _L, unroll=4)
        def _l(l):
            for g in range(_D // 16):
                v = rows[l, pl.ds(g * 16, 16)]
                plsc.store_scatter(tbuf, [colbase[g] + l], v)

        pltpu.sync_copy(tbuf, out_hbm.at[b0 + i])


def kernel(input_tensor, table):
    idx = input_tensor.astype(jnp.int32).reshape(_B * _L // _IDX_MINOR, _IDX_MINOR)
    mesh = plsc.VectorSubcoreMesh(core_axis_name="c", subcore_axis_name="s")
    run = pl.kernel(
        _tec_body,
        out_type=jax.ShapeDtypeStruct((_B, _D * _L), jnp.float32),
        mesh=mesh,
        compiler_params=pltpu.CompilerParams(needs_layout_passes=False,
                                             use_tc_tiling_on_sc=False),
        scratch_types=[
            pltpu.VMEM((_PER_W * _CHUNKS, _IDX_MINOR), jnp.int32),
            pltpu.VMEM((_L, _D), jnp.float32),
            pltpu.VMEM((_D * _L,), jnp.float32),
            pltpu.SemaphoreType.DMA,
        ],
    )
    out = run(idx, table)
    return out.reshape(_B, _D, _L)
